# baseline (speedup 1.0000x reference)
import jax
import jax.numpy as jnp
from jax.experimental import pallas as pl
from jax.experimental.pallas import tpu as pltpu

_P = jax.lax.Precision.HIGHEST


def _dot(a, b):
    return jax.lax.dot_general(
        a, b, (((1,), (0,)), ((), ())),
        preferred_element_type=jnp.float32, precision=_P)


def _stage1(gene_ref, x_ref, ws_ref, bs_ref, wf_ref, hf_ref):
    ax = _dot(gene_ref[...], x_ref[...])
    h = jnp.tanh(_dot(ax, ws_ref[...]) + bs_ref[...])
    hf_ref[...] = _dot(h, wf_ref[...])


def _stage2(adj_ref, hf_ref, bf_ref, w1_ref, b1_ref, w2_ref, b2_ref,
            w3_ref, b3_ref, out_ref):
    acc = _dot(adj_ref[...], hf_ref[...])
    h = jnp.tanh(acc + bf_ref[...])
    h = jnp.tanh(_dot(h, w1_ref[...]) + b1_ref[...])
    h = jnp.tanh(_dot(h, w2_ref[...]) + b2_ref[...])
    out_ref[...] = _dot(h, w3_ref[...]) + b3_ref[...]


def _pick_bm(n):
    for bm in (400, 256, 200, 128, 100, 80, 40, 8):
        if n % bm == 0:
            return bm
    return n


def kernel(x, adj, gene_adj, W_s, b_s, W_f, b_f, W1, b1, W2, b2, W3, b3):
    n, f = x.shape
    f1 = W1.shape[1]
    f2 = W2.shape[1]
    nc = W3.shape[1]
    bm = _pick_bm(n)
    grid = (n // bm,)
    cparams = pltpu.CompilerParams(
        dimension_semantics=("arbitrary",),
        vmem_limit_bytes=110 * 1024 * 1024,
    )

    def _const(shape):
        return pl.BlockSpec(shape, lambda i: (0, 0))

    hf = pl.pallas_call(
        _stage1,
        grid=grid,
        in_specs=[
            pl.BlockSpec((bm, n), lambda i: (i, 0)),
            _const((n, f)),
            _const((f, f)),
            _const((1, f)),
            _const((f, f)),
        ],
        out_specs=pl.BlockSpec((bm, f), lambda i: (i, 0)),
        out_shape=jax.ShapeDtypeStruct((n, f), jnp.float32),
        compiler_params=cparams,
    )(gene_adj, x, W_s, b_s.reshape(1, f), W_f)

    out = pl.pallas_call(
        _stage2,
        grid=grid,
        in_specs=[
            pl.BlockSpec((bm, n), lambda i: (i, 0)),
            _const((n, f)),
            _const((1, f)),
            _const((f, f1)),
            _const((1, f1)),
            _const((f1, f2)),
            _const((1, f2)),
            _const((f2, nc)),
            _const((1, nc)),
        ],
        out_specs=pl.BlockSpec((bm, nc), lambda i: (i, 0)),
        out_shape=jax.ShapeDtypeStruct((n, nc), jnp.float32),
        compiler_params=cparams,
    )(adj, hf, b_f.reshape(1, f), W1, b1.reshape(1, f1),
      W2, b2.reshape(1, f2), W3, b3.reshape(1, nc))
    return out


# speedup vs baseline: 2.8549x; 2.8549x over previous
import jax
import jax.numpy as jnp
from jax.experimental import pallas as pl
from jax.experimental.pallas import tpu as pltpu

_P = jax.lax.Precision.DEFAULT


def _dot(a, b):
    return jax.lax.dot_general(
        a, b, (((1,), (0,)), ((), ())),
        preferred_element_type=jnp.float32, precision=_P)


def _stage1(gene_ref, x_ref, ws_ref, bs_ref, wf_ref, hf_ref):
    ax = _dot(gene_ref[...], x_ref[...])
    h = jnp.tanh(_dot(ax, ws_ref[...]) + bs_ref[...])
    hf_ref[...] = _dot(h, wf_ref[...])


def _stage2(adj_ref, hf_ref, bf_ref, w1_ref, b1_ref, w2_ref, b2_ref,
            w3_ref, b3_ref, out_ref):
    acc = _dot(adj_ref[...], hf_ref[...])
    h = jnp.tanh(acc + bf_ref[...])
    h = jnp.tanh(_dot(h, w1_ref[...]) + b1_ref[...])
    h = jnp.tanh(_dot(h, w2_ref[...]) + b2_ref[...])
    out_ref[...] = _dot(h, w3_ref[...]) + b3_ref[...]


def _pick_bm(n):
    for bm in (400, 256, 200, 128, 100, 80, 40, 8):
        if n % bm == 0:
            return bm
    return n


def kernel(x, adj, gene_adj, W_s, b_s, W_f, b_f, W1, b1, W2, b2, W3, b3):
    n, f = x.shape
    f1 = W1.shape[1]
    f2 = W2.shape[1]
    nc = W3.shape[1]
    bm = _pick_bm(n)
    grid = (n // bm,)
    cparams = pltpu.CompilerParams(
        dimension_semantics=("arbitrary",),
        vmem_limit_bytes=110 * 1024 * 1024,
    )

    def _const(shape):
        return pl.BlockSpec(shape, lambda i: (0, 0))

    hf = pl.pallas_call(
        _stage1,
        grid=grid,
        in_specs=[
            pl.BlockSpec((bm, n), lambda i: (i, 0)),
            _const((n, f)),
            _const((f, f)),
            _const((1, f)),
            _const((f, f)),
        ],
        out_specs=pl.BlockSpec((bm, f), lambda i: (i, 0)),
        out_shape=jax.ShapeDtypeStruct((n, f), jnp.float32),
        compiler_params=cparams,
    )(gene_adj, x, W_s, b_s.reshape(1, f), W_f)

    out = pl.pallas_call(
        _stage2,
        grid=grid,
        in_specs=[
            pl.BlockSpec((bm, n), lambda i: (i, 0)),
            _const((n, f)),
            _const((1, f)),
            _const((f, f1)),
            _const((1, f1)),
            _const((f1, f2)),
            _const((1, f2)),
            _const((f2, nc)),
            _const((1, nc)),
        ],
        out_specs=pl.BlockSpec((bm, nc), lambda i: (i, 0)),
        out_shape=jax.ShapeDtypeStruct((n, nc), jnp.float32),
        compiler_params=cparams,
    )(adj, hf, b_f.reshape(1, f), W1, b1.reshape(1, f1),
      W2, b2.reshape(1, f2), W3, b3.reshape(1, nc))
    return out
